# Initial kernel scaffold; baseline (speedup 1.0000x reference)
import jax
import jax.numpy as jnp
from jax import lax
from jax.experimental import pallas as pl
from jax.experimental.pallas import tpu as pltpu
from jax.experimental.pallas import tpu_sc as plsc

N = 10000
E = 320000
NC = 2
NS = 16
NW = NC * NS
L = 16
CHUNK = 128
NCHUNK = 79
EPW = NCHUNK * CHUNK
E_PAD = NW * EPW
ACC_R = 10240
ZROWS = ACC_R // NS


def _sc_mesh():
    return plsc.VectorSubcoreMesh(
        core_axis_name="c", subcore_axis_name="s", num_cores=NC, num_subcores=NS
    )


def _make_sc_scatter(D, first):
    ZR = 4096 // D

    out_type = [jax.ShapeDtypeStruct((NC, ACC_R, D), jnp.float32)]
    scratch = [
        pltpu.VMEM((CHUNK,), jnp.int32),
        pltpu.VMEM((CHUNK,), jnp.int32),
        pltpu.VMEM((CHUNK, D), jnp.float32),
        pltpu.VMEM((ZR, D), jnp.float32),
        pltpu.VMEM_SHARED((ACC_R, D), jnp.float32),
        pltpu.SemaphoreType.DMA,
    ]
    if first:
        out_type += [
            jax.ShapeDtypeStruct((NC, ACC_R), jnp.float32),
            jax.ShapeDtypeStruct((E_PAD,), jnp.int32),
        ]
        scratch += [
            pltpu.VMEM((CHUNK,), jnp.int32),
            pltpu.VMEM((CHUNK,), jnp.float32),
            pltpu.VMEM((ZROWS,), jnp.float32),
            pltpu.VMEM_SHARED((ACC_R,), jnp.float32),
        ]

    def body(*refs):
        if first:
            (row_hbm, col_hbm, h_hbm, s_out, cnt_out, colp_out,
             row_v, cidx_v, rows_v, zbuf, acc, sem,
             colp_v, ones_v, zrow, cntacc) = refs
        else:
            (row_hbm, colp_hbm, h_hbm, s_out,
             row_v, cidx_v, rows_v, zbuf, acc, sem) = refs

        cid = lax.axis_index("c")
        sid = lax.axis_index("s")
        wid = cid * NS + sid
        zf = jnp.zeros((L,), jnp.float32)

        def zero_zbuf(r, carry):
            for c in range(D // L):
                zbuf[r, pl.ds(c * L, L)] = zf
            return carry

        lax.fori_loop(0, ZR, zero_zbuf, 0)

        z0 = sid * ZROWS

        def zero_acc(i, carry):
            pltpu.sync_copy(zbuf, acc.at[pl.ds(z0 + i * ZR, ZR), :])
            return carry

        lax.fori_loop(0, ZROWS // ZR, zero_acc, 0)

        if first:
            def zero_zrow(i, carry):
                zrow[pl.ds(i * L, L)] = zf
                return carry

            lax.fori_loop(0, ZROWS // L, zero_zrow, 0)
            pltpu.sync_copy(zrow, cntacc.at[pl.ds(z0, ZROWS)])
            one = jnp.ones((L,), jnp.float32)
            for i in range(CHUNK // L):
                ones_v[pl.ds(i * L, L)] = one

        plsc.subcore_barrier()

        base0 = wid * EPW

        def edge_chunk(j, carry):
            base = base0 + j * CHUNK
            pltpu.sync_copy(row_hbm.at[pl.ds(base, CHUNK)], row_v)
            if first:
                pltpu.sync_copy(col_hbm.at[pl.ds(base, CHUNK)], cidx_v)
                for i in range(CHUNK // L):
                    r16 = row_v[pl.ds(i * L, L)]
                    c16 = cidx_v[pl.ds(i * L, L)]
                    colp_v[pl.ds(i * L, L)] = jnp.where(
                        r16 == c16, jnp.int32(N), c16
                    )
                dst_idx = colp_v
                pltpu.sync_copy(colp_v, colp_out.at[pl.ds(base, CHUNK)])
            else:
                pltpu.sync_copy(colp_hbm.at[pl.ds(base, CHUNK)], cidx_v)
                dst_idx = cidx_v
            pltpu.async_copy(h_hbm.at[row_v], rows_v, sem).wait()
            pltpu.sync_copy(rows_v, acc.at[dst_idx], add=True)
            if first:
                pltpu.sync_copy(ones_v, cntacc.at[dst_idx], add=True)
            return carry

        lax.fori_loop(0, NCHUNK, edge_chunk, 0)

        plsc.subcore_barrier()

        def writeback(i, carry):
            r0 = z0 + i * ZR
            pltpu.sync_copy(
                acc.at[pl.ds(r0, ZR), :], s_out.at[cid, pl.ds(r0, ZR), :]
            )
            return carry

        lax.fori_loop(0, ZROWS // ZR, writeback, 0)
        if first:
            pltpu.sync_copy(
                cntacc.at[pl.ds(z0, ZROWS)], cnt_out.at[cid, pl.ds(z0, ZROWS)]
            )

    return pl.kernel(
        body, out_type=out_type, mesh=_sc_mesh(), scratch_types=scratch
    )


_sc_first = _make_sc_scatter(64, first=True)
_sc_d64 = _make_sc_scatter(64, first=False)
_sc_d128 = _make_sc_scatter(128, first=False)


BN = 2000
GRID = N // BN

_DN = (((1,), (1,)), ((), ()))


def _tc1(x, W1o, W1r):
    def body(x_ref, wo_ref, wr_ref, y_ref, z_ref):
        xb = x_ref[...]
        y_ref[...] = lax.dot_general(
            xb, wo_ref[...], _DN, preferred_element_type=jnp.float32
        )
        z_ref[...] = lax.dot_general(
            xb, wr_ref[...], _DN, preferred_element_type=jnp.float32
        )

    return pl.pallas_call(
        body,
        grid=(GRID,),
        in_specs=[
            pl.BlockSpec((BN, 128), lambda i: (i, 0)),
            pl.BlockSpec((64, 128), lambda i: (0, 0)),
            pl.BlockSpec((64, 128), lambda i: (0, 0)),
        ],
        out_specs=[
            pl.BlockSpec((BN, 64), lambda i: (i, 0)),
            pl.BlockSpec((BN, 64), lambda i: (i, 0)),
        ],
        out_shape=[
            jax.ShapeDtypeStruct((N, 64), jnp.float32),
            jax.ShapeDtypeStruct((N, 64), jnp.float32),
        ],
    )(x, W1o, W1r)


def _tc2(s0, s1, c0, c1, y, z1, b1):
    def body(s0_r, s1_r, c0_r, c1_r, y_r, z_r, b_r, o_r):
        deginv = 1.0 / (1.0 + c0_r[...] + c1_r[...])
        t = deginv * (s0_r[...] + s1_r[...] + y_r[...]) + z_r[...] + b_r[...]
        o_r[...] = jnp.maximum(t, 0.0)

    return pl.pallas_call(
        body,
        grid=(GRID,),
        in_specs=[
            pl.BlockSpec((BN, 64), lambda i: (i, 0)),
            pl.BlockSpec((BN, 64), lambda i: (i, 0)),
            pl.BlockSpec((BN, 1), lambda i: (i, 0)),
            pl.BlockSpec((BN, 1), lambda i: (i, 0)),
            pl.BlockSpec((BN, 64), lambda i: (i, 0)),
            pl.BlockSpec((BN, 64), lambda i: (i, 0)),
            pl.BlockSpec((1, 64), lambda i: (0, 0)),
        ],
        out_specs=pl.BlockSpec((BN, 64), lambda i: (i, 0)),
        out_shape=jax.ShapeDtypeStruct((N, 64), jnp.float32),
    )(s0, s1, c0, c1, y, z1, b1)


def _tc3(s0, s1, c0, c1, h1, W2o, W2r, b2):
    def body(s0_r, s1_r, c0_r, c1_r, h_r, wo_r, wr_r, b_r, o_r):
        deginv = 1.0 / (1.0 + c0_r[...] + c1_r[...])
        h = h_r[...]
        agg = deginv * (s0_r[...] + s1_r[...] + h)
        t = (
            lax.dot_general(agg, wo_r[...], _DN, preferred_element_type=jnp.float32)
            + lax.dot_general(h, wr_r[...], _DN, preferred_element_type=jnp.float32)
            + b_r[...]
        )
        o_r[...] = jnp.maximum(t, 0.0)

    return pl.pallas_call(
        body,
        grid=(GRID,),
        in_specs=[
            pl.BlockSpec((BN, 64), lambda i: (i, 0)),
            pl.BlockSpec((BN, 64), lambda i: (i, 0)),
            pl.BlockSpec((BN, 1), lambda i: (i, 0)),
            pl.BlockSpec((BN, 1), lambda i: (i, 0)),
            pl.BlockSpec((BN, 64), lambda i: (i, 0)),
            pl.BlockSpec((128, 64), lambda i: (0, 0)),
            pl.BlockSpec((128, 64), lambda i: (0, 0)),
            pl.BlockSpec((1, 128), lambda i: (0, 0)),
        ],
        out_specs=pl.BlockSpec((BN, 128), lambda i: (i, 0)),
        out_shape=jax.ShapeDtypeStruct((N, 128), jnp.float32),
    )(s0, s1, c0, c1, h1, W2o, W2r, b2)


def _tc4(s0, s1, c0, c1, h2, W3o, W3r, b3, Wlp, blp):
    def body(s0_r, s1_r, c0_r, c1_r, h_r, wo_r, wr_r, b_r, wl_r, bl_r, o_r):
        deginv = 1.0 / (1.0 + c0_r[...] + c1_r[...])
        h = h_r[...]
        agg = deginv * (s0_r[...] + s1_r[...] + h)
        h3 = jnp.maximum(
            lax.dot_general(agg, wo_r[...], _DN, preferred_element_type=jnp.float32)
            + lax.dot_general(h, wr_r[...], _DN, preferred_element_type=jnp.float32)
            + b_r[...],
            0.0,
        )
        t = (
            lax.dot_general(h3, wl_r[...], _DN, preferred_element_type=jnp.float32)
            + bl_r[...]
        )
        o_r[...] = 1.0 / (1.0 + jnp.exp(-t))

    return pl.pallas_call(
        body,
        grid=(GRID,),
        in_specs=[
            pl.BlockSpec((BN, 128), lambda i: (i, 0)),
            pl.BlockSpec((BN, 128), lambda i: (i, 0)),
            pl.BlockSpec((BN, 1), lambda i: (i, 0)),
            pl.BlockSpec((BN, 1), lambda i: (i, 0)),
            pl.BlockSpec((BN, 128), lambda i: (i, 0)),
            pl.BlockSpec((256, 128), lambda i: (0, 0)),
            pl.BlockSpec((256, 128), lambda i: (0, 0)),
            pl.BlockSpec((1, 256), lambda i: (0, 0)),
            pl.BlockSpec((128, 256), lambda i: (0, 0)),
            pl.BlockSpec((1, 128), lambda i: (0, 0)),
        ],
        out_specs=pl.BlockSpec((BN, 128), lambda i: (i, 0)),
        out_shape=jax.ShapeDtypeStruct((N, 128), jnp.float32),
    )(s0, s1, c0, c1, h2, W3o, W3r, b3, Wlp, blp)


def kernel(x, edge_index, batch_graph, W1o, b1, W1r, W2o, b2, W2r, W3o, b3, W3r,
           Wl, bl):
    del batch_graph
    pad = jnp.zeros((E_PAD - E,), jnp.int32)
    row_p = jnp.concatenate([edge_index[0], pad])
    col_p = jnp.concatenate([edge_index[1], pad])

    y, z1 = _tc1(x, W1o, W1r)
    s1, cnt, colp = _sc_first(row_p, col_p, y)
    c0 = cnt[0, :N, None]
    c1 = cnt[1, :N, None]
    h1 = _tc2(s1[0, :N], s1[1, :N], c0, c1, y, z1, b1[None, :])
    s2 = _sc_d64(row_p, colp, h1)
    h2 = _tc3(s2[0, :N], s2[1, :N], c0, c1, h1, W2o, W2r, b2[None, :])
    s3 = _sc_d128(row_p, colp, h2)
    Wlp = jnp.zeros((128, 256), jnp.float32).at[:6, :].set(Wl)
    blp = jnp.zeros((128,), jnp.float32).at[:6].set(bl)
    out = _tc4(s3[0, :N], s3[1, :N], c0, c1, h2, W3o, W3r, b3[None, :],
               Wlp, blp[None, :])
    return out[:, :6]


# speedup vs baseline: 9.3835x; 9.3835x over previous
import jax
import jax.numpy as jnp
from jax import lax
from jax.experimental import pallas as pl
from jax.experimental.pallas import tpu as pltpu
from jax.experimental.pallas import tpu_sc as plsc

N = 10000
E = 320000
NC = 2
NS = 16
NW = NC * NS
L = 16
CHUNK = 128
NCHUNK = 79
EPW = NCHUNK * CHUNK
E_PAD = NW * EPW
ACC_R = 10240
ZROWS = ACC_R // NS


def _sc_mesh():
    return plsc.VectorSubcoreMesh(
        core_axis_name="c", subcore_axis_name="s", num_cores=NC, num_subcores=NS
    )


def _make_sc_scatter(D, first):
    ZR = 4096 // D

    out_type = [jax.ShapeDtypeStruct((NC, ACC_R, D), jnp.float32)]
    scratch = [
        pltpu.VMEM((CHUNK,), jnp.int32),
        pltpu.VMEM((CHUNK,), jnp.int32),
        pltpu.VMEM((CHUNK, D), jnp.float32),
        pltpu.VMEM((ZR, D), jnp.float32),
        pltpu.VMEM_SHARED((ACC_R, D), jnp.float32),
        pltpu.SemaphoreType.DMA,
    ]
    if first:
        out_type += [
            jax.ShapeDtypeStruct((NC, ACC_R), jnp.float32),
            jax.ShapeDtypeStruct((E_PAD,), jnp.int32),
        ]
        scratch += [
            pltpu.VMEM((CHUNK,), jnp.int32),
            pltpu.VMEM((CHUNK,), jnp.float32),
            pltpu.VMEM((ZROWS,), jnp.float32),
            pltpu.VMEM_SHARED((ACC_R,), jnp.float32),
        ]

    def body(*refs):
        if first:
            (row_hbm, col_hbm, h_hbm, s_out, cnt_out, colp_out,
             row_v, cidx_v, rows_v, zbuf, acc, sem,
             colp_v, ones_v, zrow, cntacc) = refs
        else:
            (row_hbm, colp_hbm, h_hbm, s_out,
             row_v, cidx_v, rows_v, zbuf, acc, sem) = refs

        cid = lax.axis_index("c")
        sid = lax.axis_index("s")
        wid = cid * NS + sid
        zf = jnp.zeros((L,), jnp.float32)

        def zero_zbuf(r, carry):
            for c in range(D // L):
                zbuf[r, pl.ds(c * L, L)] = zf
            return carry

        lax.fori_loop(0, ZR, zero_zbuf, 0)

        z0 = sid * ZROWS

        def zero_acc(i, carry):
            pltpu.sync_copy(zbuf, acc.at[pl.ds(z0 + i * ZR, ZR), :])
            return carry

        lax.fori_loop(0, ZROWS // ZR, zero_acc, 0)

        if first:
            def zero_zrow(i, carry):
                zrow[pl.ds(i * L, L)] = zf
                return carry

            lax.fori_loop(0, ZROWS // L, zero_zrow, 0)
            pltpu.sync_copy(zrow, cntacc.at[pl.ds(z0, ZROWS)])
            one = jnp.ones((L,), jnp.float32)
            for i in range(CHUNK // L):
                ones_v[pl.ds(i * L, L)] = one

        plsc.subcore_barrier()

        base0 = wid * EPW

        def edge_chunk(j, carry):
            base = base0 + j * CHUNK
            pltpu.sync_copy(row_hbm.at[pl.ds(base, CHUNK)], row_v)
            if first:
                pltpu.sync_copy(col_hbm.at[pl.ds(base, CHUNK)], cidx_v)
                for i in range(CHUNK // L):
                    r16 = row_v[pl.ds(i * L, L)]
                    c16 = cidx_v[pl.ds(i * L, L)]
                    colp_v[pl.ds(i * L, L)] = jnp.where(
                        r16 == c16, jnp.int32(N), c16
                    )
                dst_idx = colp_v
                pltpu.sync_copy(colp_v, colp_out.at[pl.ds(base, CHUNK)])
            else:
                pltpu.sync_copy(colp_hbm.at[pl.ds(base, CHUNK)], cidx_v)
                dst_idx = cidx_v
            pltpu.async_copy(h_hbm.at[row_v], rows_v, sem).wait()
            pltpu.sync_copy(rows_v, acc.at[dst_idx], add=True)
            if first:
                pltpu.sync_copy(ones_v, cntacc.at[dst_idx], add=True)
            return carry

        lax.fori_loop(0, NCHUNK, edge_chunk, 0)

        plsc.subcore_barrier()

        def writeback(i, carry):
            r0 = z0 + i * ZR
            pltpu.sync_copy(
                acc.at[pl.ds(r0, ZR), :], s_out.at[cid, pl.ds(r0, ZR), :]
            )
            return carry

        lax.fori_loop(0, ZROWS // ZR, writeback, 0)
        if first:
            pltpu.sync_copy(
                cntacc.at[pl.ds(z0, ZROWS)], cnt_out.at[cid, pl.ds(z0, ZROWS)]
            )

    return pl.kernel(
        body,
        out_type=out_type,
        mesh=_sc_mesh(),
        scratch_types=scratch,
        compiler_params=pltpu.CompilerParams(use_tc_tiling_on_sc=False),
    )


_sc_first = _make_sc_scatter(64, first=True)
_sc_d64 = _make_sc_scatter(64, first=False)
_sc_d128 = _make_sc_scatter(128, first=False)


BN = 2000
GRID = N // BN

_DN = (((1,), (1,)), ((), ()))


def _tc1(x, W1o, W1r):
    def body(x_ref, wo_ref, wr_ref, y_ref, z_ref):
        xb = x_ref[...]
        y_ref[...] = lax.dot_general(
            xb, wo_ref[...], _DN, preferred_element_type=jnp.float32
        )
        z_ref[...] = lax.dot_general(
            xb, wr_ref[...], _DN, preferred_element_type=jnp.float32
        )

    return pl.pallas_call(
        body,
        grid=(GRID,),
        in_specs=[
            pl.BlockSpec((BN, 128), lambda i: (i, 0)),
            pl.BlockSpec((64, 128), lambda i: (0, 0)),
            pl.BlockSpec((64, 128), lambda i: (0, 0)),
        ],
        out_specs=[
            pl.BlockSpec((BN, 64), lambda i: (i, 0)),
            pl.BlockSpec((BN, 64), lambda i: (i, 0)),
        ],
        out_shape=[
            jax.ShapeDtypeStruct((N, 64), jnp.float32),
            jax.ShapeDtypeStruct((N, 64), jnp.float32),
        ],
    )(x, W1o, W1r)


def _tc2(s0, s1, c0, c1, y, z1, b1):
    def body(s0_r, s1_r, c0_r, c1_r, y_r, z_r, b_r, o_r):
        deginv = 1.0 / (1.0 + c0_r[...] + c1_r[...])
        t = deginv * (s0_r[...] + s1_r[...] + y_r[...]) + z_r[...] + b_r[...]
        o_r[...] = jnp.maximum(t, 0.0)

    return pl.pallas_call(
        body,
        grid=(GRID,),
        in_specs=[
            pl.BlockSpec((BN, 64), lambda i: (i, 0)),
            pl.BlockSpec((BN, 64), lambda i: (i, 0)),
            pl.BlockSpec((BN, 1), lambda i: (i, 0)),
            pl.BlockSpec((BN, 1), lambda i: (i, 0)),
            pl.BlockSpec((BN, 64), lambda i: (i, 0)),
            pl.BlockSpec((BN, 64), lambda i: (i, 0)),
            pl.BlockSpec((1, 64), lambda i: (0, 0)),
        ],
        out_specs=pl.BlockSpec((BN, 64), lambda i: (i, 0)),
        out_shape=jax.ShapeDtypeStruct((N, 64), jnp.float32),
    )(s0, s1, c0, c1, y, z1, b1)


def _tc3(s0, s1, c0, c1, h1, W2o, W2r, b2):
    def body(s0_r, s1_r, c0_r, c1_r, h_r, wo_r, wr_r, b_r, o_r):
        deginv = 1.0 / (1.0 + c0_r[...] + c1_r[...])
        h = h_r[...]
        agg = deginv * (s0_r[...] + s1_r[...] + h)
        t = (
            lax.dot_general(agg, wo_r[...], _DN, preferred_element_type=jnp.float32)
            + lax.dot_general(h, wr_r[...], _DN, preferred_element_type=jnp.float32)
            + b_r[...]
        )
        o_r[...] = jnp.maximum(t, 0.0)

    return pl.pallas_call(
        body,
        grid=(GRID,),
        in_specs=[
            pl.BlockSpec((BN, 64), lambda i: (i, 0)),
            pl.BlockSpec((BN, 64), lambda i: (i, 0)),
            pl.BlockSpec((BN, 1), lambda i: (i, 0)),
            pl.BlockSpec((BN, 1), lambda i: (i, 0)),
            pl.BlockSpec((BN, 64), lambda i: (i, 0)),
            pl.BlockSpec((128, 64), lambda i: (0, 0)),
            pl.BlockSpec((128, 64), lambda i: (0, 0)),
            pl.BlockSpec((1, 128), lambda i: (0, 0)),
        ],
        out_specs=pl.BlockSpec((BN, 128), lambda i: (i, 0)),
        out_shape=jax.ShapeDtypeStruct((N, 128), jnp.float32),
    )(s0, s1, c0, c1, h1, W2o, W2r, b2)


def _tc4(s0, s1, c0, c1, h2, W3o, W3r, b3, Wlp, blp):
    def body(s0_r, s1_r, c0_r, c1_r, h_r, wo_r, wr_r, b_r, wl_r, bl_r, o_r):
        deginv = 1.0 / (1.0 + c0_r[...] + c1_r[...])
        h = h_r[...]
        agg = deginv * (s0_r[...] + s1_r[...] + h)
        h3 = jnp.maximum(
            lax.dot_general(agg, wo_r[...], _DN, preferred_element_type=jnp.float32)
            + lax.dot_general(h, wr_r[...], _DN, preferred_element_type=jnp.float32)
            + b_r[...],
            0.0,
        )
        t = (
            lax.dot_general(h3, wl_r[...], _DN, preferred_element_type=jnp.float32)
            + bl_r[...]
        )
        o_r[...] = 1.0 / (1.0 + jnp.exp(-t))

    return pl.pallas_call(
        body,
        grid=(GRID,),
        in_specs=[
            pl.BlockSpec((BN, 128), lambda i: (i, 0)),
            pl.BlockSpec((BN, 128), lambda i: (i, 0)),
            pl.BlockSpec((BN, 1), lambda i: (i, 0)),
            pl.BlockSpec((BN, 1), lambda i: (i, 0)),
            pl.BlockSpec((BN, 128), lambda i: (i, 0)),
            pl.BlockSpec((256, 128), lambda i: (0, 0)),
            pl.BlockSpec((256, 128), lambda i: (0, 0)),
            pl.BlockSpec((1, 256), lambda i: (0, 0)),
            pl.BlockSpec((128, 256), lambda i: (0, 0)),
            pl.BlockSpec((1, 128), lambda i: (0, 0)),
        ],
        out_specs=pl.BlockSpec((BN, 128), lambda i: (i, 0)),
        out_shape=jax.ShapeDtypeStruct((N, 128), jnp.float32),
    )(s0, s1, c0, c1, h2, W3o, W3r, b3, Wlp, blp)


def kernel(x, edge_index, batch_graph, W1o, b1, W1r, W2o, b2, W2r, W3o, b3, W3r,
           Wl, bl):
    del batch_graph
    pad = jnp.zeros((E_PAD - E,), jnp.int32)
    row_p = jnp.concatenate([edge_index[0], pad])
    col_p = jnp.concatenate([edge_index[1], pad])

    y, z1 = _tc1(x, W1o, W1r)
    s1, cnt, colp = _sc_first(row_p, col_p, y)
    c0 = cnt[0, :N, None]
    c1 = cnt[1, :N, None]
    h1 = _tc2(s1[0, :N], s1[1, :N], c0, c1, y, z1, b1[None, :])
    (s2,) = _sc_d64(row_p, colp, h1)
    h2 = _tc3(s2[0, :N], s2[1, :N], c0, c1, h1, W2o, W2r, b2[None, :])
    (s3,) = _sc_d128(row_p, colp, h2)
    Wlp = jnp.zeros((128, 256), jnp.float32).at[:6, :].set(Wl)
    blp = jnp.zeros((128,), jnp.float32).at[:6].set(bl)
    out = _tc4(s3[0, :N], s3[1, :N], c0, c1, h2, W3o, W3r, b3[None, :],
               Wlp, blp[None, :])
    return out[:, :6]
